# baseline (speedup 1.0000x reference)
import functools

import jax
import jax.numpy as jnp
from jax import lax
from jax.experimental import pallas as pl
from jax.experimental.pallas import tpu as pltpu
from jax.experimental.pallas import tpu_sc as plsc

_N = 100000
_D = 128
_B = 10000
_K = 32
_EMB = 64

_NC = 2
_NS = 16
_NW = _NC * _NS
_BPW = 320
_BPAD = _NW * _BPW
_IDXROWS = _BPW * _K // 128
_EPS = 1e-5

_R0 = 320
_R1 = 320
_BAND = _R0 + _R1
_IDXSTAGE = max(_R0, _R1) * _K // 128

_INV_K = 1.0 / _K
_INV_B = 1.0 / _B


def _accum_chunk(gbuf, obuf, orow_base):
    def node_body(j, c3):
        row = j * _K

        def k_body(k, accs):
            r0 = row + k * 2
            return tuple(
                accs[s_] + (gbuf[r0, pl.ds(s_ * 16, 16)]
                            + gbuf[r0 + 1, pl.ds(s_ * 16, 16)])
                for s_ in range(8))

        zero = jnp.zeros((16,), jnp.float32)
        accs = lax.fori_loop(0, _K // 2, k_body, (zero,) * 8)
        for s_ in range(8):
            obuf[orow_base + j, pl.ds(s_ * 16, 16)] = accs[s_] * _INV_K
        return c3

    lax.fori_loop(0, 4, node_body, 0)


def _sc_body(feat, feat16, nodes_v, nbr_all, s_out, a_out,
             idx_v, nidx_v, gb0, gb1, ob0, sb0, sb1,
             gs0, gs1, ws0, ws1):
    wid = lax.axis_index("s") * _NC + lax.axis_index("c")
    base = wid * _BPW
    gbufs = (gb0, gb1)
    gsems = (gs0, gs1)
    ob = ob0
    wsems = (ws0, ws1)
    sbufs = (sb0, sb1)

    pltpu.sync_copy(nodes_v.at[pl.ds(wid * 8, 8)], nidx_v)
    for i in range(2):
        pltpu.async_copy(feat.at[nidx_v.at[i]], sbufs[i], gsems[i])
    for i in range(8):
        h = i % 2
        dst = s_out.at[pl.ds(base + i * 40, 40)]
        pltpu.make_async_copy(feat.at[nidx_v.at[i]], sbufs[h], gsems[h]).wait()
        pltpu.async_copy(sbufs[h], dst, wsems[h])
        if i + 2 < 8:
            pltpu.make_async_copy(sbufs[h], dst, wsems[h]).wait()
            pltpu.async_copy(feat.at[nidx_v.at[i + 2]], sbufs[h], gsems[h])
    for i in range(6, 8):
        h = i % 2
        pltpu.make_async_copy(
            sbufs[h], s_out.at[pl.ds(base + i * 40, 40)], wsems[h]).wait()

    cc = lax.axis_index("c")
    sid = lax.axis_index("s")
    nbase = sid * _BAND + cc * _R0
    idxbase = sid * (_BAND * _K // 128) + cc * (_R0 * _K // 128)
    nrows = jnp.where(cc == 0, _R0, _R1)
    nchunks = nrows // 4
    nq = nrows // 8

    def rel_body(r, carry):
        @pl.when(nq > 0)
        def _():
            pltpu.sync_copy(
                nbr_all.at[r, pl.ds(idxbase, _IDXSTAGE)], idx_v)
            for h in range(2):
                pltpu.async_copy(feat16.at[idx_v.at[h]], gbufs[h], gsems[h])

            def q_body(q, c2):
                @pl.when(q > 0)
                def _():
                    pltpu.make_async_copy(
                        ob, a_out.at[r, pl.ds(nbase, 8)], wsems[0]).wait()
                for h in range(2):
                    g = q * 2 + h
                    pltpu.make_async_copy(
                        feat16.at[idx_v.at[h]], gbufs[h], gsems[h]).wait()
                    _accum_chunk(gbufs[h], ob, 4 * h)
                    @pl.when(g + 2 < nchunks)
                    def _():
                        pltpu.async_copy(
                            feat16.at[idx_v.at[g + 2]], gbufs[h], gsems[h])
                pltpu.async_copy(
                    ob, a_out.at[r, pl.ds(nbase + q * 8, 8)], wsems[0])
                return c2

            lax.fori_loop(0, nq, q_body, 0)
            pltpu.make_async_copy(
                ob, a_out.at[r, pl.ds(nbase, 8)], wsems[0]).wait()

        return carry

    lax.fori_loop(0, 3, rel_body, 0)


_sc_call = pl.kernel(
    _sc_body,
    out_type=[
        jax.ShapeDtypeStruct((_BPAD, _D), jnp.float32),
        jax.ShapeDtypeStruct((3, _BPAD, _D), jnp.float32),
    ],
    mesh=plsc.VectorSubcoreMesh(
        core_axis_name="c", subcore_axis_name="s",
        num_cores=_NC, num_subcores=_NS),
    scratch_types=[
        pltpu.VMEM((_IDXSTAGE, 128), jnp.int32),
        pltpu.VMEM((8, 40), jnp.int32),
        pltpu.VMEM((128, _D), jnp.float32),
        pltpu.VMEM((128, _D), jnp.float32),
        pltpu.VMEM((8, _D), jnp.float32),
        pltpu.VMEM((40, _D), jnp.float32),
        pltpu.VMEM((40, _D), jnp.float32),
        pltpu.SemaphoreType.DMA,
        pltpu.SemaphoreType.DMA,
        pltpu.SemaphoreType.DMA,
        pltpu.SemaphoreType.DMA,
    ],
)


_SBS = 2000


def _stats_body(s_ref, a1_ref, a2_ref, a3_ref, o_ref):
    i = pl.program_id(0)
    s = s_ref[...]
    a1 = a1_ref[...]
    a2 = a2_ref[...]
    a3 = a3_ref[...]
    rows = [
        jnp.sum(s, 0), jnp.sum(s * s, 0),
        jnp.sum(a1, 0), jnp.sum(a1 * a1, 0), jnp.sum(s * a1, 0),
        jnp.sum(a2, 0), jnp.sum(a2 * a2, 0), jnp.sum(s * a2, 0),
        jnp.sum(a3, 0), jnp.sum(a3 * a3, 0), jnp.sum(s * a3, 0),
    ]
    z = jnp.zeros_like(rows[0])
    blk = jnp.stack(rows + [z] * 5)

    @pl.when(i == 0)
    def _():
        o_ref[...] = blk

    @pl.when(i > 0)
    def _():
        o_ref[...] += blk


_stats_call = pl.pallas_call(
    _stats_body,
    grid=(_B // _SBS,),
    in_specs=[pl.BlockSpec((_SBS, _D), lambda i: (i, 0))] * 4,
    out_specs=pl.BlockSpec((16, _D), lambda i: (0, 0)),
    out_shape=jax.ShapeDtypeStruct((16, _D), jnp.float32),
)


_MBS = 1024


def _mm_body(st_ref, w_ref, g_ref, b_ref, s_ref, a1_ref, a2_ref, a3_ref, o_ref):
    st = st_ref[...]
    mS = st[0] * _INV_B
    vS = st[1] * _INV_B - mS * mS
    W = w_ref[...]
    gb = g_ref[...]
    bb = b_ref[...]

    means = [None] * 10
    varis = [None] * 10
    for j in (0, 1, 4, 7):
        means[j] = mS
        varis[j] = vS
    for r in range(3):
        mA = st[2 + 3 * r] * _INV_B
        vA = st[3 + 3 * r] * _INV_B - mA * mA
        cv = st[4 + 3 * r] * _INV_B - mS * mA
        means[2 + 3 * r] = mA
        varis[2 + 3 * r] = vA
        means[3 + 3 * r] = mS - mA
        varis[3 + 3 * r] = vS + vA - 2.0 * cv

    sc = [gb[j] * lax.rsqrt(varis[j] + _EPS) for j in range(10)]
    WS = sum(sc[j][:, None] * W[j] for j in (0, 1, 3, 4, 6, 7, 9))
    WA = [sc[2 + 3 * r][:, None] * W[2 + 3 * r]
          - sc[3 + 3 * r][:, None] * W[3 + 3 * r] for r in range(3)]
    Wcat = jnp.concatenate([WS] + WA, axis=0)
    coef = jnp.stack([bb[j] - means[j] * sc[j] for j in range(10)])
    bias = jax.lax.dot_general(
        coef.reshape(1, 10 * _D), w_ref[...].reshape(10 * _D, _EMB),
        (((1,), (0,)), ((), ())), preferred_element_type=jnp.float32)

    X = jnp.concatenate(
        [s_ref[...], a1_ref[...], a2_ref[...], a3_ref[...]], axis=1)
    y = jnp.dot(X, Wcat, preferred_element_type=jnp.float32) + bias
    o_ref[...] = jnp.maximum(y, 0.0).T


_mm_call = pl.pallas_call(
    _mm_body,
    grid=(_BPAD // _MBS,),
    in_specs=[
        pl.BlockSpec((16, _D), lambda i: (0, 0)),
        pl.BlockSpec((10, _D, _EMB), lambda i: (0, 0, 0)),
        pl.BlockSpec((10, _D), lambda i: (0, 0)),
        pl.BlockSpec((10, _D), lambda i: (0, 0)),
        pl.BlockSpec((_MBS, _D), lambda i: (i, 0)),
        pl.BlockSpec((_MBS, _D), lambda i: (i, 0)),
        pl.BlockSpec((_MBS, _D), lambda i: (i, 0)),
        pl.BlockSpec((_MBS, _D), lambda i: (i, 0)),
    ],
    out_specs=pl.BlockSpec((_EMB, _MBS), lambda i: (0, i)),
    out_shape=jax.ShapeDtypeStruct((_EMB, _BPAD), jnp.float32),
)


def kernel(features, nodes, labels, neighbors_r1, neighbors_r2, neighbors_r3,
           weight1, gamma, beta):
    del labels
    pad = _BPAD - _B
    nodes_p = jnp.concatenate(
        [nodes.astype(jnp.int32), jnp.zeros((pad,), jnp.int32)]
    ).reshape(_NW * 8, 40)
    idx_rows = _BPAD * _K // 128
    idx_rows_pad = 2688
    nbr_rows = []
    for nb in (neighbors_r1, neighbors_r2, neighbors_r3):
        nbp = jnp.concatenate(
            [nb.astype(jnp.int32), jnp.zeros((pad, _K), jnp.int32)], axis=0)
        nbp = nbp.reshape(idx_rows, 128)
        nbp = jnp.concatenate(
            [nbp, jnp.zeros((idx_rows_pad - idx_rows, 128), jnp.int32)], axis=0)
        nbr_rows.append(nbp)
    nbr_all = jnp.stack(nbr_rows)

    s_pad, a_pad = _sc_call(features, features, nodes_p, nbr_all)

    st = _stats_call(s_pad[:_B], a_pad[0, :_B], a_pad[1, :_B], a_pad[2, :_B])

    out = _mm_call(st, weight1.reshape(10, _D, _EMB), gamma.reshape(10, _D),
                   beta.reshape(10, _D), s_pad, a_pad[0], a_pad[1], a_pad[2])
    return out[:, :_B]


# speedup vs baseline: 1.0758x; 1.0758x over previous
import functools

import jax
import jax.numpy as jnp
from jax import lax
from jax.experimental import pallas as pl
from jax.experimental.pallas import tpu as pltpu
from jax.experimental.pallas import tpu_sc as plsc

_N = 100000
_D = 128
_B = 10000
_K = 32
_EMB = 64

_NC = 2
_NS = 16
_NW = _NC * _NS
_BPW = 320
_BPAD = _NW * _BPW
_IDXROWS = _BPW * _K // 128
_EPS = 1e-5

_R0 = 512
_R1 = 128
_BAND = _R0 + _R1
_IDXSTAGE = max(_R0, _R1) * _K // 128

_INV_K = 1.0 / _K
_INV_B = 1.0 / _B


def _accum_chunk(gbuf, obuf, orow_base):
    def node_body(j, c3):
        row = j * _K
        for s_ in range(_D // 16):
            sl = pl.ds(s_ * 16, 16)
            acc = None
            for k in range(0, _K, 4):
                qd = ((gbuf[row + k, sl] + gbuf[row + k + 1, sl])
                      + (gbuf[row + k + 2, sl] + gbuf[row + k + 3, sl]))
                acc = qd if acc is None else acc + qd
            obuf[orow_base + j, sl] = acc * _INV_K
        return c3

    lax.fori_loop(0, 4, node_body, 0)


def _sc_body(feat, nodes_v, nbr_all, s_out, a_out,
             idx_v, nidx_v, gb0, gb1, gb2, gb3, ob0, ob1, sb0, sb1,
             gs0, gs1, gs2, gs3, ws0, ws1):
    wid = lax.axis_index("s") * _NC + lax.axis_index("c")
    base = wid * _BPW
    gbufs = (gb0, gb1, gb2, gb3)
    gsems = (gs0, gs1, gs2, gs3)
    obufs = (ob0, ob1)
    wsems = (ws0, ws1)
    sbufs = (sb0, sb1)

    pltpu.sync_copy(nodes_v.at[pl.ds(wid * 8, 8)], nidx_v)
    for i in range(2):
        pltpu.async_copy(feat.at[nidx_v.at[i]], sbufs[i], gsems[i])
    for i in range(8):
        h = i % 2
        dst = s_out.at[pl.ds(base + i * 40, 40)]
        pltpu.make_async_copy(feat.at[nidx_v.at[i]], sbufs[h], gsems[h]).wait()
        pltpu.async_copy(sbufs[h], dst, wsems[h])
        if i + 2 < 8:
            pltpu.make_async_copy(sbufs[h], dst, wsems[h]).wait()
            pltpu.async_copy(feat.at[nidx_v.at[i + 2]], sbufs[h], gsems[h])
    for i in range(6, 8):
        h = i % 2
        pltpu.make_async_copy(
            sbufs[h], s_out.at[pl.ds(base + i * 40, 40)], wsems[h]).wait()

    cc = lax.axis_index("c")
    sid = lax.axis_index("s")
    nbase = sid * _BAND + cc * _R0
    idxbase = sid * (_BAND * _K // 128) + cc * (_R0 * _K // 128)
    nrows = jnp.where(cc == 0, _R0, _R1)
    nchunks = nrows // 4
    nq = nrows // 32

    def rel_body(r, carry):
        @pl.when(nq > 0)
        def _():
            pltpu.sync_copy(
                nbr_all.at[r, pl.ds(idxbase, _IDXSTAGE)], idx_v)
            for h in range(4):
                pltpu.async_copy(feat.at[idx_v.at[h]], gbufs[h], gsems[h])

            def q_body(q, c2):
                for t in range(2):
                    ob = obufs[t]
                    @pl.when(q > 0)
                    def _():
                        pltpu.make_async_copy(
                            ob, a_out.at[r, pl.ds(nbase, 16)],
                            wsems[t]).wait()
                    for h in range(4):
                        g = q * 8 + t * 4 + h
                        pltpu.make_async_copy(
                            feat.at[idx_v.at[h]], gbufs[h], gsems[h]).wait()
                        _accum_chunk(gbufs[h], ob, 4 * h)
                        @pl.when(g + 4 < nchunks)
                        def _():
                            pltpu.async_copy(
                                feat.at[idx_v.at[g + 4]], gbufs[h], gsems[h])
                    pltpu.async_copy(
                        ob, a_out.at[r, pl.ds(nbase + (q * 2 + t) * 16, 16)],
                        wsems[t])
                return c2

            lax.fori_loop(0, nq, q_body, 0)
            for t in range(2):
                pltpu.make_async_copy(
                    obufs[t], a_out.at[r, pl.ds(nbase, 16)], wsems[t]).wait()

        return carry

    lax.fori_loop(0, 3, rel_body, 0)


_sc_call = pl.kernel(
    _sc_body,
    out_type=[
        jax.ShapeDtypeStruct((_BPAD, _D), jnp.float32),
        jax.ShapeDtypeStruct((3, _BPAD, _D), jnp.float32),
    ],
    mesh=plsc.VectorSubcoreMesh(
        core_axis_name="c", subcore_axis_name="s",
        num_cores=_NC, num_subcores=_NS),
    scratch_types=[
        pltpu.VMEM((_IDXSTAGE, 128), jnp.int32),
        pltpu.VMEM((8, 40), jnp.int32),
        pltpu.VMEM((128, _D), jnp.float32),
        pltpu.VMEM((128, _D), jnp.float32),
        pltpu.VMEM((128, _D), jnp.float32),
        pltpu.VMEM((128, _D), jnp.float32),
        pltpu.VMEM((16, _D), jnp.float32),
        pltpu.VMEM((16, _D), jnp.float32),
        pltpu.VMEM((40, _D), jnp.float32),
        pltpu.VMEM((40, _D), jnp.float32),
        pltpu.SemaphoreType.DMA,
        pltpu.SemaphoreType.DMA,
        pltpu.SemaphoreType.DMA,
        pltpu.SemaphoreType.DMA,
        pltpu.SemaphoreType.DMA,
        pltpu.SemaphoreType.DMA,
    ],
)


_SBS = 2000


def _stats_body(s_ref, a1_ref, a2_ref, a3_ref, o_ref):
    i = pl.program_id(0)
    s = s_ref[...]
    a1 = a1_ref[...]
    a2 = a2_ref[...]
    a3 = a3_ref[...]
    rows = [
        jnp.sum(s, 0), jnp.sum(s * s, 0),
        jnp.sum(a1, 0), jnp.sum(a1 * a1, 0), jnp.sum(s * a1, 0),
        jnp.sum(a2, 0), jnp.sum(a2 * a2, 0), jnp.sum(s * a2, 0),
        jnp.sum(a3, 0), jnp.sum(a3 * a3, 0), jnp.sum(s * a3, 0),
    ]
    z = jnp.zeros_like(rows[0])
    blk = jnp.stack(rows + [z] * 5)

    @pl.when(i == 0)
    def _():
        o_ref[...] = blk

    @pl.when(i > 0)
    def _():
        o_ref[...] += blk


_stats_call = pl.pallas_call(
    _stats_body,
    grid=(_B // _SBS,),
    in_specs=[pl.BlockSpec((_SBS, _D), lambda i: (i, 0))] * 4,
    out_specs=pl.BlockSpec((16, _D), lambda i: (0, 0)),
    out_shape=jax.ShapeDtypeStruct((16, _D), jnp.float32),
)


_MBS = 1024


def _mm_body(st_ref, w_ref, g_ref, b_ref, s_ref, a1_ref, a2_ref, a3_ref, o_ref):
    st = st_ref[...]
    mS = st[0] * _INV_B
    vS = st[1] * _INV_B - mS * mS
    W = w_ref[...]
    gb = g_ref[...]
    bb = b_ref[...]

    means = [None] * 10
    varis = [None] * 10
    for j in (0, 1, 4, 7):
        means[j] = mS
        varis[j] = vS
    for r in range(3):
        mA = st[2 + 3 * r] * _INV_B
        vA = st[3 + 3 * r] * _INV_B - mA * mA
        cv = st[4 + 3 * r] * _INV_B - mS * mA
        means[2 + 3 * r] = mA
        varis[2 + 3 * r] = vA
        means[3 + 3 * r] = mS - mA
        varis[3 + 3 * r] = vS + vA - 2.0 * cv

    sc = [gb[j] * lax.rsqrt(varis[j] + _EPS) for j in range(10)]
    WS = sum(sc[j][:, None] * W[j] for j in (0, 1, 3, 4, 6, 7, 9))
    WA = [sc[2 + 3 * r][:, None] * W[2 + 3 * r]
          - sc[3 + 3 * r][:, None] * W[3 + 3 * r] for r in range(3)]
    Wcat = jnp.concatenate([WS] + WA, axis=0)
    coef = jnp.stack([bb[j] - means[j] * sc[j] for j in range(10)])
    bias = jax.lax.dot_general(
        coef.reshape(1, 10 * _D), w_ref[...].reshape(10 * _D, _EMB),
        (((1,), (0,)), ((), ())), preferred_element_type=jnp.float32)

    X = jnp.concatenate(
        [s_ref[...], a1_ref[...], a2_ref[...], a3_ref[...]], axis=1)
    y = jnp.dot(X, Wcat, preferred_element_type=jnp.float32) + bias
    o_ref[...] = jnp.maximum(y, 0.0).T


_mm_call = pl.pallas_call(
    _mm_body,
    grid=(_BPAD // _MBS,),
    in_specs=[
        pl.BlockSpec((16, _D), lambda i: (0, 0)),
        pl.BlockSpec((10, _D, _EMB), lambda i: (0, 0, 0)),
        pl.BlockSpec((10, _D), lambda i: (0, 0)),
        pl.BlockSpec((10, _D), lambda i: (0, 0)),
        pl.BlockSpec((_MBS, _D), lambda i: (i, 0)),
        pl.BlockSpec((_MBS, _D), lambda i: (i, 0)),
        pl.BlockSpec((_MBS, _D), lambda i: (i, 0)),
        pl.BlockSpec((_MBS, _D), lambda i: (i, 0)),
    ],
    out_specs=pl.BlockSpec((_EMB, _MBS), lambda i: (0, i)),
    out_shape=jax.ShapeDtypeStruct((_EMB, _BPAD), jnp.float32),
)


def kernel(features, nodes, labels, neighbors_r1, neighbors_r2, neighbors_r3,
           weight1, gamma, beta):
    del labels
    pad = _BPAD - _B
    nodes_p = jnp.concatenate(
        [nodes.astype(jnp.int32), jnp.zeros((pad,), jnp.int32)]
    ).reshape(_NW * 8, 40)
    idx_rows = _BPAD * _K // 128
    idx_rows_pad = 2688
    nbr_rows = []
    for nb in (neighbors_r1, neighbors_r2, neighbors_r3):
        nbp = jnp.concatenate(
            [nb.astype(jnp.int32), jnp.zeros((pad, _K), jnp.int32)], axis=0)
        nbp = nbp.reshape(idx_rows, 128)
        nbp = jnp.concatenate(
            [nbp, jnp.zeros((idx_rows_pad - idx_rows, 128), jnp.int32)], axis=0)
        nbr_rows.append(nbp)
    nbr_all = jnp.stack(nbr_rows)

    s_pad, a_pad = _sc_call(features, nodes_p, nbr_all)

    st = _stats_call(s_pad[:_B], a_pad[0, :_B], a_pad[1, :_B], a_pad[2, :_B])

    out = _mm_call(st, weight1.reshape(10, _D, _EMB), gamma.reshape(10, _D),
                   beta.reshape(10, _D), s_pad, a_pad[0], a_pad[1], a_pad[2])
    return out[:, :_B]
